# baseline (speedup 1.0000x reference)
import functools

import jax
import jax.numpy as jnp
from jax import lax
from jax.experimental import pallas as pl
from jax.experimental.pallas import tpu as pltpu
from jax.experimental.pallas import tpu_sc as plsc

N = 10000
E = 320000
DIN = 128
H = 256
HH = 128
G = 64
A = 16

NC = 2
NT = 16
K = 80
ER = E // K
CH = ER // NT
DCH = ER // (NC * NT)
RPT = 624

BLK = 1000
NBLK = N // BLK


def _mesh():
    return plsc.VectorSubcoreMesh(core_axis_name="c", subcore_axis_name="s",
                                  num_cores=NC, num_subcores=NT)


def _deg_body(e2, deg0, deg1, acc, ones, didx, sem):
    del sem
    c = lax.axis_index("c")
    s = lax.axis_index("s")

    zv = jnp.zeros((16,), jnp.float32)

    def zr(r, _):
        ones[r, pl.ds(0, 16)] = zv
        return 0

    lax.fori_loop(0, K, zr, 0)
    z0 = s * RPT
    for i in range(7):
        pltpu.sync_copy(ones, acc.at[pl.ds(z0 + i * K, K)])
    pltpu.sync_copy(ones.at[pl.ds(0, 64)], acc.at[pl.ds(z0 + 7 * K, 64)])

    @pl.when(s == NT - 1)
    def _():
        pltpu.sync_copy(ones.at[pl.ds(0, 16)], acc.at[pl.ds(NT * RPT, 16)])

    ev = jnp.where(lax.iota(jnp.int32, 16) == 0, 1.0, 0.0).astype(jnp.float32)

    def er(r, _):
        ones[r, pl.ds(0, 16)] = ev
        return 0

    lax.fori_loop(0, K, er, 0)

    dbase = c * (NT * DCH) + s * 128
    cnt = jnp.where(s == NT - 1, NT * DCH - 15 * 128, 128)

    @pl.when(s < NT - 1)
    def _():
        pltpu.sync_copy(e2.at[1, pl.ds(dbase, 128)], didx)

    @pl.when(s == NT - 1)
    def _():
        pltpu.sync_copy(e2.at[1, pl.ds(dbase, NT * DCH - 15 * 128)],
                        didx.at[pl.ds(0, NT * DCH - 15 * 128)])

    plsc.subcore_barrier()

    def body(j, _):
        pltpu.sync_copy(ones, acc.at[didx.at[j]], add=True)
        return 0

    lax.fori_loop(0, cnt, body, 0)

    plsc.subcore_barrier()

    r0 = s * RPT

    @pl.when(c == 0)
    def _():
        pltpu.sync_copy(acc.at[pl.ds(r0, RPT)], deg0.at[pl.ds(r0, RPT)])

        @pl.when(s == NT - 1)
        def _():
            pltpu.sync_copy(acc.at[pl.ds(NT * RPT, 16)],
                            deg0.at[pl.ds(NT * RPT, 16)])

    @pl.when(c == 1)
    def _():
        pltpu.sync_copy(acc.at[pl.ds(r0, RPT)], deg1.at[pl.ds(r0, RPT)])

        @pl.when(s == NT - 1)
        def _():
            pltpu.sync_copy(acc.at[pl.ds(NT * RPT, 16)],
                            deg1.at[pl.ds(NT * RPT, 16)])


@functools.cache
def _deg_call():
    return pl.kernel(
        _deg_body,
        out_type=[jax.ShapeDtypeStruct((N, 16), jnp.float32),
                  jax.ShapeDtypeStruct((N, 16), jnp.float32)],
        mesh=_mesh(),
        scratch_types=[
            pltpu.VMEM_SHARED((N, 16), jnp.float32),
            pltpu.VMEM((K, 16), jnp.float32),
            pltpu.VMEM((128, K), jnp.int32),
            pltpu.SemaphoreType.DMA,
        ],
    )


def _agg_body(e2, p_lo, p_hi, agg_lo, agg_hi, acc, idx, rows,
              sg0, sg1, sg2, sg3, si0, si1, sem_s):
    sems = (sg0, sg1, sg2, sg3)
    isems = (si0, si1)
    c = lax.axis_index("c")
    s = lax.axis_index("s")
    rbase = s * CH

    zv = jnp.zeros((16,), jnp.float32)

    def zr(r, _):
        for k2 in range(HH // 16):
            rows[0, r, pl.ds(k2 * 16, 16)] = zv
        return 0

    lax.fori_loop(0, K, zr, 0)
    z0 = s * RPT
    for i in range(7):
        pltpu.sync_copy(rows.at[0], acc.at[pl.ds(z0 + i * K, K)])
    pltpu.sync_copy(rows.at[0, pl.ds(0, 64)], acc.at[pl.ds(z0 + 7 * K, 64)])

    @pl.when(s == NT - 1)
    def _():
        pltpu.sync_copy(rows.at[0, pl.ds(0, 16)], acc.at[pl.ds(NT * RPT, 16)])

    plsc.subcore_barrier()

    def fire_idx(j, b, ps):
        pltpu.async_copy(e2.at[0, rbase + j], idx.at[b, 0], isems[ps])
        pltpu.async_copy(e2.at[1, rbase + j], idx.at[b, 1], isems[ps])

    def wait_idx(j, b, ps):
        pltpu.make_async_copy(e2.at[0, rbase + j], idx.at[b, 0],
                              isems[ps]).wait()
        pltpu.make_async_copy(e2.at[1, rbase + j], idx.at[b, 1],
                              isems[ps]).wait()

    def fire(b, rb):
        @pl.when(c == 0)
        def _():
            pltpu.async_copy(p_lo.at[idx.at[b, 0]], rows.at[rb], sems[rb])

        @pl.when(c == 1)
        def _():
            pltpu.async_copy(p_hi.at[idx.at[b, 0]], rows.at[rb], sems[rb])

    def wait(b, rb):
        @pl.when(c == 0)
        def _():
            pltpu.make_async_copy(p_lo.at[idx.at[b, 0]], rows.at[rb],
                                  sems[rb]).wait()

        @pl.when(c == 1)
        def _():
            pltpu.make_async_copy(p_hi.at[idx.at[b, 0]], rows.at[rb],
                                  sems[rb]).wait()

    def fire_scat(b, rb):
        pltpu.async_copy(rows.at[rb], acc.at[idx.at[b, 1]], sem_s, add=True)

    def wait_scat(b, rb):
        pltpu.make_async_copy(rows.at[rb], acc.at[idx.at[b, 1]],
                              sem_s).wait()

    pltpu.sync_copy(e2.at[0, rbase], idx.at[0, 0])
    pltpu.sync_copy(e2.at[1, rbase], idx.at[0, 1])
    pltpu.sync_copy(e2.at[0, rbase + 1], idx.at[1, 0])
    pltpu.sync_copy(e2.at[1, rbase + 1], idx.at[1, 1])
    pltpu.sync_copy(e2.at[0, rbase + 2], idx.at[2, 0])
    pltpu.sync_copy(e2.at[1, rbase + 2], idx.at[2, 1])
    fire_idx(3, 3, 1)
    fire_idx(4, 4, 0)
    fire(0, 0)
    fire(1, 1)
    fire(2, 2)

    def body(j8, _):
        for u in range(8):
            j = j8 * 8 + u

            @pl.when(j + 3 < CH)
            def _():
                wait_idx(j + 3, (u + 3) % 8, (u + 1) % 2)

            @pl.when(j < CH)
            def _():
                wait(u % 8, u % 4)

            @pl.when((j - 1 >= 0) & (j - 1 < CH))
            def _():
                wait_scat((u + 7) % 8, (u + 3) % 4)

            @pl.when(j + 3 < CH)
            def _():
                fire((u + 3) % 8, (u + 3) % 4)

            @pl.when(j < CH)
            def _():
                fire_scat(u % 8, u % 4)

            @pl.when(j + 5 < CH)
            def _():
                fire_idx(j + 5, (u + 5) % 8, (u + 1) % 2)

        return 0

    lax.fori_loop(0, (CH + 7) // 8, body, 0)

    plsc.subcore_barrier()

    r0 = s * RPT

    @pl.when(c == 0)
    def _():
        pltpu.sync_copy(acc.at[pl.ds(r0, RPT)], agg_lo.at[pl.ds(r0, RPT)])

        @pl.when(s == NT - 1)
        def _():
            pltpu.sync_copy(acc.at[pl.ds(NT * RPT, 16)],
                            agg_lo.at[pl.ds(NT * RPT, 16)])

    @pl.when(c == 1)
    def _():
        pltpu.sync_copy(acc.at[pl.ds(r0, RPT)], agg_hi.at[pl.ds(r0, RPT)])

        @pl.when(s == NT - 1)
        def _():
            pltpu.sync_copy(acc.at[pl.ds(NT * RPT, 16)],
                            agg_hi.at[pl.ds(NT * RPT, 16)])


@functools.cache
def _agg_call():
    return pl.kernel(
        _agg_body,
        out_type=[jax.ShapeDtypeStruct((N, HH), jnp.float32),
                  jax.ShapeDtypeStruct((N, HH), jnp.float32)],
        mesh=_mesh(),
        scratch_types=[
            pltpu.VMEM_SHARED((N, HH), jnp.float32),
            pltpu.VMEM((8, 2, K), jnp.int32),
            pltpu.VMEM((4, K, HH), jnp.float32),
            pltpu.SemaphoreType.DMA,
            pltpu.SemaphoreType.DMA,
            pltpu.SemaphoreType.DMA,
            pltpu.SemaphoreType.DMA,
            pltpu.SemaphoreType.DMA,
            pltpu.SemaphoreType.DMA,
            pltpu.SemaphoreType.DMA,
        ],
    )


def _tcmm_body(x, W1, xw):
    xw[...] = jnp.dot(x[...], W1[...], preferred_element_type=jnp.float32)


def _tcmm(x, W1):
    return pl.pallas_call(
        _tcmm_body,
        grid=(NBLK,),
        in_specs=[
            pl.BlockSpec((BLK, DIN), lambda i: (i, 0)),
            pl.BlockSpec((DIN, H), lambda i: (0, 0)),
        ],
        out_specs=pl.BlockSpec((BLK, H), lambda i: (i, 0)),
        out_shape=jax.ShapeDtypeStruct((N, H), jnp.float32),
    )(x, W1)


def _tc1_body(deg0, deg1, xw, plo, phi, dinv):
    deg = (jnp.sum(deg0[...], axis=1, keepdims=True)
           + jnp.sum(deg1[...], axis=1, keepdims=True) + 1.0)
    dv = lax.rsqrt(deg)
    p = xw[...] * dv
    plo[...] = p[:, :HH]
    phi[...] = p[:, HH:]
    dinv[...] = jnp.broadcast_to(dv, (BLK, 8))


def _tc1(deg0, deg1, xw):
    return pl.pallas_call(
        _tc1_body,
        grid=(NBLK,),
        in_specs=[
            pl.BlockSpec((BLK, 16), lambda i: (i, 0)),
            pl.BlockSpec((BLK, 16), lambda i: (i, 0)),
            pl.BlockSpec((BLK, H), lambda i: (i, 0)),
        ],
        out_specs=[
            pl.BlockSpec((BLK, HH), lambda i: (i, 0)),
            pl.BlockSpec((BLK, HH), lambda i: (i, 0)),
            pl.BlockSpec((BLK, 8), lambda i: (i, 0)),
        ],
        out_shape=[
            jax.ShapeDtypeStruct((N, HH), jnp.float32),
            jax.ShapeDtypeStruct((N, HH), jnp.float32),
            jax.ShapeDtypeStruct((N, 8), jnp.float32),
        ],
    )(deg0, deg1, xw)


def _tc2_body(alo, ahi, plo, phi, dinv, b1, W2, olo, ohi):
    dv = dinv[...][:, :1]
    h = jnp.concatenate([alo[...] + plo[...], ahi[...] + phi[...]], axis=1)
    h = jnp.maximum(h * dv + b1[0:1, :], 0.0)
    p2 = jnp.dot(h, W2[...], preferred_element_type=jnp.float32) * dv
    olo[...] = p2[:, :HH]
    ohi[...] = p2[:, HH:]


def _tc2(alo, ahi, plo, phi, dinv, b1r, W2):
    return pl.pallas_call(
        _tc2_body,
        grid=(NBLK,),
        in_specs=[
            pl.BlockSpec((BLK, HH), lambda i: (i, 0)),
            pl.BlockSpec((BLK, HH), lambda i: (i, 0)),
            pl.BlockSpec((BLK, HH), lambda i: (i, 0)),
            pl.BlockSpec((BLK, HH), lambda i: (i, 0)),
            pl.BlockSpec((BLK, 8), lambda i: (i, 0)),
            pl.BlockSpec((8, H), lambda i: (0, 0)),
            pl.BlockSpec((H, H), lambda i: (0, 0)),
        ],
        out_specs=[
            pl.BlockSpec((BLK, HH), lambda i: (i, 0)),
            pl.BlockSpec((BLK, HH), lambda i: (i, 0)),
        ],
        out_shape=[
            jax.ShapeDtypeStruct((N, HH), jnp.float32),
            jax.ShapeDtypeStruct((N, HH), jnp.float32),
        ],
    )(alo, ahi, plo, phi, dinv, b1r, W2)


def _tc3_body(alo, ahi, plo, phi, dinv, b2, batch3,
              fW1, fb1, fW2, fb2, aW1, ab1, aW2, ab2, vW1, vb1, vW2, vb2,
              probs_o, val_o, gsum, cnt):
    i = pl.program_id(0)

    @pl.when(i == 0)
    def _():
        gsum[...] = jnp.zeros_like(gsum)
        cnt[...] = jnp.zeros_like(cnt)

    dv = dinv[...][:, :1]
    h = jnp.concatenate([alo[...] + plo[...], ahi[...] + phi[...]], axis=1)
    h = jnp.maximum(h * dv + b2[0:1, :], 0.0)

    bb = batch3[...].reshape(1, BLK)
    oh = (lax.broadcasted_iota(jnp.int32, (G, BLK), 0) == bb).astype(jnp.float32)
    gsum[...] += jnp.dot(oh, h, preferred_element_type=jnp.float32)
    cnt[...] += jnp.broadcast_to(jnp.sum(oh, axis=1, keepdims=True), (G, 8))

    @pl.when(i == NBLK - 1)
    def _():
        g = gsum[...] / jnp.maximum(cnt[...][:, :1], 1.0)
        g = jnp.maximum(
            jnp.dot(g, fW1[...], preferred_element_type=jnp.float32) + fb1[0:1, :], 0.0)
        g = jnp.maximum(
            jnp.dot(g, fW2[...], preferred_element_type=jnp.float32) + fb2[0:1, :], 0.0)
        adv = jnp.maximum(
            jnp.dot(g, aW1[...], preferred_element_type=jnp.float32) + ab1[0:1, :], 0.0)
        adv = jnp.dot(adv, aW2[...], preferred_element_type=jnp.float32) + ab2[0:1, :]
        v = jnp.maximum(
            jnp.dot(g, vW1[...], preferred_element_type=jnp.float32) + vb1[0:1, :], 0.0)
        v8 = jnp.dot(v, vW2[...], preferred_element_type=jnp.float32) + vb2[0:1, :]
        logits = adv + v8[:, :1] - jnp.mean(adv, axis=1, keepdims=True)
        m = jnp.max(logits, axis=1, keepdims=True)
        e = jnp.exp(logits - m)
        probs_o[...] = e / jnp.sum(e, axis=1, keepdims=True)
        val_o[...] = v8


def _tc3(alo, ahi, plo, phi, dinv, b2r, batch3, ws):
    def wspec(r, c):
        return pl.BlockSpec((r, c), lambda i: (0, 0))

    return pl.pallas_call(
        _tc3_body,
        grid=(NBLK,),
        in_specs=[
            pl.BlockSpec((BLK, HH), lambda i: (i, 0)),
            pl.BlockSpec((BLK, HH), lambda i: (i, 0)),
            pl.BlockSpec((BLK, HH), lambda i: (i, 0)),
            pl.BlockSpec((BLK, HH), lambda i: (i, 0)),
            pl.BlockSpec((BLK, 8), lambda i: (i, 0)),
            pl.BlockSpec((8, H), lambda i: (0, 0)),
            pl.BlockSpec((1, 1, BLK), lambda i: (i, 0, 0)),
            wspec(H, H), wspec(8, H),
            wspec(H, H), wspec(8, H),
            wspec(H, H), wspec(8, H),
            wspec(H, A), wspec(8, A),
            wspec(H, H), wspec(8, H),
            wspec(H, 8), wspec(8, 8),
        ],
        out_specs=[
            pl.BlockSpec((G, A), lambda i: (0, 0)),
            pl.BlockSpec((G, 8), lambda i: (0, 0)),
        ],
        out_shape=[
            jax.ShapeDtypeStruct((G, A), jnp.float32),
            jax.ShapeDtypeStruct((G, 8), jnp.float32),
        ],
        scratch_shapes=[
            pltpu.VMEM((G, H), jnp.float32),
            pltpu.VMEM((G, 8), jnp.float32),
        ],
    )(alo, ahi, plo, phi, dinv, b2r, batch3, *ws)


def kernel(x, edge_index, batch, W1, b1, W2, b2, fW1, fb1, fW2, fb2,
           aW1, ab1, aW2, ab2, vW1, vb1, vW2, vb2):
    e2 = edge_index.reshape(2, ER, K)
    batch3 = batch.reshape(NBLK, 1, BLK)
    b1r = jnp.broadcast_to(b1[None, :], (8, H))
    b2r = jnp.broadcast_to(b2[None, :], (8, H))
    fb1r = jnp.broadcast_to(fb1[None, :], (8, H))
    fb2r = jnp.broadcast_to(fb2[None, :], (8, H))
    ab1r = jnp.broadcast_to(ab1[None, :], (8, H))
    ab2r = jnp.broadcast_to(ab2[None, :], (8, A))
    vb1r = jnp.broadcast_to(vb1[None, :], (8, H))
    vW2p = jnp.broadcast_to(vW2, (H, 8))
    vb2p = jnp.broadcast_to(vb2[None, :], (8, 8))

    xw1 = _tcmm(x, W1)
    deg0, deg1 = _deg_call()(e2)
    p1_lo, p1_hi, dinv = _tc1(deg0, deg1, xw1)
    agg1_lo, agg1_hi = _agg_call()(e2, p1_lo, p1_hi)
    p2_lo, p2_hi = _tc2(agg1_lo, agg1_hi, p1_lo, p1_hi, dinv, b1r, W2)
    agg2_lo, agg2_hi = _agg_call()(e2, p2_lo, p2_hi)
    ws = (fW1, fb1r, fW2, fb2r, aW1, ab1r, aW2, ab2r, vW1, vb1r, vW2p, vb2p)
    probs, v8 = _tc3(agg2_lo, agg2_hi, p2_lo, p2_hi, dinv, b2r, batch3, ws)
    return (probs, v8[:, :1])


# speedup vs baseline: 1.0212x; 1.0212x over previous
import functools

import jax
import jax.numpy as jnp
from jax import lax
from jax.experimental import pallas as pl
from jax.experimental.pallas import tpu as pltpu
from jax.experimental.pallas import tpu_sc as plsc

N = 10000
E = 320000
DIN = 128
H = 256
HH = 128
G = 64
A = 16

NC = 2
NT = 16
K = 80
ER = E // K
CH = ER // NT
DCH = ER // (NC * NT)
RPT = 624

BLK = 1000
NBLK = N // BLK


def _mesh():
    return plsc.VectorSubcoreMesh(core_axis_name="c", subcore_axis_name="s",
                                  num_cores=NC, num_subcores=NT)


def _deg_body(e2, deg0, deg1, acc, ones, didx, sem):
    c = lax.axis_index("c")
    s = lax.axis_index("s")

    zv = jnp.zeros((16,), jnp.float32)

    def zr(r, _):
        ones[r, pl.ds(0, 16)] = zv
        return 0

    lax.fori_loop(0, K, zr, 0)
    z0 = s * RPT
    for i in range(7):
        pltpu.sync_copy(ones, acc.at[pl.ds(z0 + i * K, K)])
    pltpu.sync_copy(ones.at[pl.ds(0, 64)], acc.at[pl.ds(z0 + 7 * K, 64)])

    @pl.when(s == NT - 1)
    def _():
        pltpu.sync_copy(ones.at[pl.ds(0, 16)], acc.at[pl.ds(NT * RPT, 16)])

    ev = jnp.where(lax.iota(jnp.int32, 16) == 0, 1.0, 0.0).astype(jnp.float32)

    def er(r, _):
        ones[r, pl.ds(0, 16)] = ev
        return 0

    lax.fori_loop(0, K, er, 0)

    dbase = c * (NT * DCH) + s * 128
    cnt = jnp.where(s == NT - 1, NT * DCH - 15 * 128, 128)

    @pl.when(s < NT - 1)
    def _():
        pltpu.sync_copy(e2.at[1, pl.ds(dbase, 128)], didx)

    @pl.when(s == NT - 1)
    def _():
        pltpu.sync_copy(e2.at[1, pl.ds(dbase, NT * DCH - 15 * 128)],
                        didx.at[pl.ds(0, NT * DCH - 15 * 128)])

    plsc.subcore_barrier()

    def body(j, _):
        pltpu.async_copy(ones, acc.at[didx.at[j]], sem, add=True)

        @pl.when(j >= 4)
        def _():
            pltpu.make_async_copy(ones, acc.at[didx.at[j - 4]], sem).wait()

        return 0

    lax.fori_loop(0, cnt, body, 0)

    def drain(j, _):
        pltpu.make_async_copy(ones, acc.at[didx.at[j]], sem).wait()
        return 0

    lax.fori_loop(cnt - 4, cnt, drain, 0)

    plsc.subcore_barrier()

    r0 = s * RPT

    @pl.when(c == 0)
    def _():
        pltpu.sync_copy(acc.at[pl.ds(r0, RPT)], deg0.at[pl.ds(r0, RPT)])

        @pl.when(s == NT - 1)
        def _():
            pltpu.sync_copy(acc.at[pl.ds(NT * RPT, 16)],
                            deg0.at[pl.ds(NT * RPT, 16)])

    @pl.when(c == 1)
    def _():
        pltpu.sync_copy(acc.at[pl.ds(r0, RPT)], deg1.at[pl.ds(r0, RPT)])

        @pl.when(s == NT - 1)
        def _():
            pltpu.sync_copy(acc.at[pl.ds(NT * RPT, 16)],
                            deg1.at[pl.ds(NT * RPT, 16)])


@functools.cache
def _deg_call():
    return pl.kernel(
        _deg_body,
        out_type=[jax.ShapeDtypeStruct((N, 16), jnp.float32),
                  jax.ShapeDtypeStruct((N, 16), jnp.float32)],
        mesh=_mesh(),
        scratch_types=[
            pltpu.VMEM_SHARED((N, 16), jnp.float32),
            pltpu.VMEM((K, 16), jnp.float32),
            pltpu.VMEM((128, K), jnp.int32),
            pltpu.SemaphoreType.DMA,
        ],
    )


def _agg_body(e2, p_lo, p_hi, agg_lo, agg_hi, acc, idx, rows,
              sg0, sg1, sg2, sg3, si0, si1, sem_s):
    sems = (sg0, sg1, sg2, sg3)
    isems = (si0, si1)
    c = lax.axis_index("c")
    s = lax.axis_index("s")
    rbase = s * CH

    zv = jnp.zeros((16,), jnp.float32)

    def zr(r, _):
        for k2 in range(HH // 16):
            rows[0, r, pl.ds(k2 * 16, 16)] = zv
        return 0

    lax.fori_loop(0, K, zr, 0)
    z0 = s * RPT
    for i in range(7):
        pltpu.sync_copy(rows.at[0], acc.at[pl.ds(z0 + i * K, K)])
    pltpu.sync_copy(rows.at[0, pl.ds(0, 64)], acc.at[pl.ds(z0 + 7 * K, 64)])

    @pl.when(s == NT - 1)
    def _():
        pltpu.sync_copy(rows.at[0, pl.ds(0, 16)], acc.at[pl.ds(NT * RPT, 16)])

    plsc.subcore_barrier()

    def fire_idx(j, b, ps):
        pltpu.async_copy(e2.at[0, rbase + j], idx.at[b, 0], isems[ps])
        pltpu.async_copy(e2.at[1, rbase + j], idx.at[b, 1], isems[ps])

    def wait_idx(j, b, ps):
        pltpu.make_async_copy(e2.at[0, rbase + j], idx.at[b, 0],
                              isems[ps]).wait()
        pltpu.make_async_copy(e2.at[1, rbase + j], idx.at[b, 1],
                              isems[ps]).wait()

    def fire(b, rb):
        @pl.when(c == 0)
        def _():
            pltpu.async_copy(p_lo.at[idx.at[b, 0]], rows.at[rb], sems[rb])

        @pl.when(c == 1)
        def _():
            pltpu.async_copy(p_hi.at[idx.at[b, 0]], rows.at[rb], sems[rb])

    def wait(b, rb):
        @pl.when(c == 0)
        def _():
            pltpu.make_async_copy(p_lo.at[idx.at[b, 0]], rows.at[rb],
                                  sems[rb]).wait()

        @pl.when(c == 1)
        def _():
            pltpu.make_async_copy(p_hi.at[idx.at[b, 0]], rows.at[rb],
                                  sems[rb]).wait()

    def fire_scat(b, rb):
        pltpu.async_copy(rows.at[rb], acc.at[idx.at[b, 1]], sem_s, add=True)

    def wait_scat(b, rb):
        pltpu.make_async_copy(rows.at[rb], acc.at[idx.at[b, 1]],
                              sem_s).wait()

    pltpu.sync_copy(e2.at[0, rbase], idx.at[0, 0])
    pltpu.sync_copy(e2.at[1, rbase], idx.at[0, 1])
    pltpu.sync_copy(e2.at[0, rbase + 1], idx.at[1, 0])
    pltpu.sync_copy(e2.at[1, rbase + 1], idx.at[1, 1])
    pltpu.sync_copy(e2.at[0, rbase + 2], idx.at[2, 0])
    pltpu.sync_copy(e2.at[1, rbase + 2], idx.at[2, 1])
    fire_idx(3, 3, 1)
    fire_idx(4, 4, 0)
    fire(0, 0)
    fire(1, 1)
    fire(2, 2)

    def body(j8, _):
        for u in range(8):
            j = j8 * 8 + u

            @pl.when(j + 3 < CH)
            def _():
                wait_idx(j + 3, (u + 3) % 8, (u + 1) % 2)

            @pl.when(j < CH)
            def _():
                wait(u % 8, u % 4)

            @pl.when((j - 1 >= 0) & (j - 1 < CH))
            def _():
                wait_scat((u + 7) % 8, (u + 3) % 4)

            @pl.when(j + 3 < CH)
            def _():
                fire((u + 3) % 8, (u + 3) % 4)

            @pl.when(j < CH)
            def _():
                fire_scat(u % 8, u % 4)

            @pl.when(j + 5 < CH)
            def _():
                fire_idx(j + 5, (u + 5) % 8, (u + 1) % 2)

        return 0

    lax.fori_loop(0, (CH + 7) // 8, body, 0)

    plsc.subcore_barrier()

    r0 = s * RPT

    @pl.when(c == 0)
    def _():
        pltpu.sync_copy(acc.at[pl.ds(r0, RPT)], agg_lo.at[pl.ds(r0, RPT)])

        @pl.when(s == NT - 1)
        def _():
            pltpu.sync_copy(acc.at[pl.ds(NT * RPT, 16)],
                            agg_lo.at[pl.ds(NT * RPT, 16)])

    @pl.when(c == 1)
    def _():
        pltpu.sync_copy(acc.at[pl.ds(r0, RPT)], agg_hi.at[pl.ds(r0, RPT)])

        @pl.when(s == NT - 1)
        def _():
            pltpu.sync_copy(acc.at[pl.ds(NT * RPT, 16)],
                            agg_hi.at[pl.ds(NT * RPT, 16)])


@functools.cache
def _agg_call():
    return pl.kernel(
        _agg_body,
        out_type=[jax.ShapeDtypeStruct((N, HH), jnp.float32),
                  jax.ShapeDtypeStruct((N, HH), jnp.float32)],
        mesh=_mesh(),
        scratch_types=[
            pltpu.VMEM_SHARED((N, HH), jnp.float32),
            pltpu.VMEM((8, 2, K), jnp.int32),
            pltpu.VMEM((4, K, HH), jnp.float32),
            pltpu.SemaphoreType.DMA,
            pltpu.SemaphoreType.DMA,
            pltpu.SemaphoreType.DMA,
            pltpu.SemaphoreType.DMA,
            pltpu.SemaphoreType.DMA,
            pltpu.SemaphoreType.DMA,
            pltpu.SemaphoreType.DMA,
        ],
    )


def _tc1_body(deg0, deg1, x, W1, plo, phi, dinv):
    deg = (jnp.sum(deg0[...], axis=1, keepdims=True)
           + jnp.sum(deg1[...], axis=1, keepdims=True) + 1.0)
    dv = lax.rsqrt(deg)
    p = jnp.dot(x[...], W1[...], preferred_element_type=jnp.float32) * dv
    plo[...] = p[:, :HH]
    phi[...] = p[:, HH:]
    dinv[...] = jnp.broadcast_to(dv, (BLK, 8))


def _tc1(deg0, deg1, x, W1):
    return pl.pallas_call(
        _tc1_body,
        grid=(NBLK,),
        in_specs=[
            pl.BlockSpec((BLK, 16), lambda i: (i, 0)),
            pl.BlockSpec((BLK, 16), lambda i: (i, 0)),
            pl.BlockSpec((BLK, DIN), lambda i: (i, 0)),
            pl.BlockSpec((DIN, H), lambda i: (0, 0)),
        ],
        out_specs=[
            pl.BlockSpec((BLK, HH), lambda i: (i, 0)),
            pl.BlockSpec((BLK, HH), lambda i: (i, 0)),
            pl.BlockSpec((BLK, 8), lambda i: (i, 0)),
        ],
        out_shape=[
            jax.ShapeDtypeStruct((N, HH), jnp.float32),
            jax.ShapeDtypeStruct((N, HH), jnp.float32),
            jax.ShapeDtypeStruct((N, 8), jnp.float32),
        ],
    )(deg0, deg1, x, W1)


def _tc2_body(alo, ahi, plo, phi, dinv, b1, W2, olo, ohi):
    dv = dinv[...][:, :1]
    h = jnp.concatenate([alo[...] + plo[...], ahi[...] + phi[...]], axis=1)
    h = jnp.maximum(h * dv + b1[0:1, :], 0.0)
    p2 = jnp.dot(h, W2[...], preferred_element_type=jnp.float32) * dv
    olo[...] = p2[:, :HH]
    ohi[...] = p2[:, HH:]


def _tc2(alo, ahi, plo, phi, dinv, b1r, W2):
    return pl.pallas_call(
        _tc2_body,
        grid=(NBLK,),
        in_specs=[
            pl.BlockSpec((BLK, HH), lambda i: (i, 0)),
            pl.BlockSpec((BLK, HH), lambda i: (i, 0)),
            pl.BlockSpec((BLK, HH), lambda i: (i, 0)),
            pl.BlockSpec((BLK, HH), lambda i: (i, 0)),
            pl.BlockSpec((BLK, 8), lambda i: (i, 0)),
            pl.BlockSpec((8, H), lambda i: (0, 0)),
            pl.BlockSpec((H, H), lambda i: (0, 0)),
        ],
        out_specs=[
            pl.BlockSpec((BLK, HH), lambda i: (i, 0)),
            pl.BlockSpec((BLK, HH), lambda i: (i, 0)),
        ],
        out_shape=[
            jax.ShapeDtypeStruct((N, HH), jnp.float32),
            jax.ShapeDtypeStruct((N, HH), jnp.float32),
        ],
    )(alo, ahi, plo, phi, dinv, b1r, W2)


def _tc3_body(alo, ahi, plo, phi, dinv, b2, batch3,
              fW1, fb1, fW2, fb2, aW1, ab1, aW2, ab2, vW1, vb1, vW2, vb2,
              probs_o, val_o, gsum, cnt):
    i = pl.program_id(0)

    @pl.when(i == 0)
    def _():
        gsum[...] = jnp.zeros_like(gsum)
        cnt[...] = jnp.zeros_like(cnt)

    dv = dinv[...][:, :1]
    h = jnp.concatenate([alo[...] + plo[...], ahi[...] + phi[...]], axis=1)
    h = jnp.maximum(h * dv + b2[0:1, :], 0.0)

    bb = batch3[...].reshape(1, BLK)
    oh = (lax.broadcasted_iota(jnp.int32, (G, BLK), 0) == bb).astype(jnp.float32)
    gsum[...] += jnp.dot(oh, h, preferred_element_type=jnp.float32)
    cnt[...] += jnp.broadcast_to(jnp.sum(oh, axis=1, keepdims=True), (G, 8))

    @pl.when(i == NBLK - 1)
    def _():
        g = gsum[...] / jnp.maximum(cnt[...][:, :1], 1.0)
        g = jnp.maximum(
            jnp.dot(g, fW1[...], preferred_element_type=jnp.float32) + fb1[0:1, :], 0.0)
        g = jnp.maximum(
            jnp.dot(g, fW2[...], preferred_element_type=jnp.float32) + fb2[0:1, :], 0.0)
        adv = jnp.maximum(
            jnp.dot(g, aW1[...], preferred_element_type=jnp.float32) + ab1[0:1, :], 0.0)
        adv = jnp.dot(adv, aW2[...], preferred_element_type=jnp.float32) + ab2[0:1, :]
        v = jnp.maximum(
            jnp.dot(g, vW1[...], preferred_element_type=jnp.float32) + vb1[0:1, :], 0.0)
        v8 = jnp.dot(v, vW2[...], preferred_element_type=jnp.float32) + vb2[0:1, :]
        logits = adv + v8[:, :1] - jnp.mean(adv, axis=1, keepdims=True)
        m = jnp.max(logits, axis=1, keepdims=True)
        e = jnp.exp(logits - m)
        probs_o[...] = e / jnp.sum(e, axis=1, keepdims=True)
        val_o[...] = v8


def _tc3(alo, ahi, plo, phi, dinv, b2r, batch3, ws):
    def wspec(r, c):
        return pl.BlockSpec((r, c), lambda i: (0, 0))

    return pl.pallas_call(
        _tc3_body,
        grid=(NBLK,),
        in_specs=[
            pl.BlockSpec((BLK, HH), lambda i: (i, 0)),
            pl.BlockSpec((BLK, HH), lambda i: (i, 0)),
            pl.BlockSpec((BLK, HH), lambda i: (i, 0)),
            pl.BlockSpec((BLK, HH), lambda i: (i, 0)),
            pl.BlockSpec((BLK, 8), lambda i: (i, 0)),
            pl.BlockSpec((8, H), lambda i: (0, 0)),
            pl.BlockSpec((1, 1, BLK), lambda i: (i, 0, 0)),
            wspec(H, H), wspec(8, H),
            wspec(H, H), wspec(8, H),
            wspec(H, H), wspec(8, H),
            wspec(H, A), wspec(8, A),
            wspec(H, H), wspec(8, H),
            wspec(H, 8), wspec(8, 8),
        ],
        out_specs=[
            pl.BlockSpec((G, A), lambda i: (0, 0)),
            pl.BlockSpec((G, 8), lambda i: (0, 0)),
        ],
        out_shape=[
            jax.ShapeDtypeStruct((G, A), jnp.float32),
            jax.ShapeDtypeStruct((G, 8), jnp.float32),
        ],
        scratch_shapes=[
            pltpu.VMEM((G, H), jnp.float32),
            pltpu.VMEM((G, 8), jnp.float32),
        ],
    )(alo, ahi, plo, phi, dinv, b2r, batch3, *ws)


def kernel(x, edge_index, batch, W1, b1, W2, b2, fW1, fb1, fW2, fb2,
           aW1, ab1, aW2, ab2, vW1, vb1, vW2, vb2):
    e2 = edge_index.reshape(2, ER, K)
    batch3 = batch.reshape(NBLK, 1, BLK)
    b1r = jnp.broadcast_to(b1[None, :], (8, H))
    b2r = jnp.broadcast_to(b2[None, :], (8, H))
    fb1r = jnp.broadcast_to(fb1[None, :], (8, H))
    fb2r = jnp.broadcast_to(fb2[None, :], (8, H))
    ab1r = jnp.broadcast_to(ab1[None, :], (8, H))
    ab2r = jnp.broadcast_to(ab2[None, :], (8, A))
    vb1r = jnp.broadcast_to(vb1[None, :], (8, H))
    vW2p = jnp.broadcast_to(vW2, (H, 8))
    vb2p = jnp.broadcast_to(vb2[None, :], (8, 8))

    deg0, deg1 = _deg_call()(e2)
    p1_lo, p1_hi, dinv = _tc1(deg0, deg1, x, W1)
    agg1_lo, agg1_hi = _agg_call()(e2, p1_lo, p1_hi)
    p2_lo, p2_hi = _tc2(agg1_lo, agg1_hi, p1_lo, p1_hi, dinv, b1r, W2)
    agg2_lo, agg2_hi = _agg_call()(e2, p2_lo, p2_hi)
    ws = (fW1, fb1r, fW2, fb2r, aW1, ab1r, aW2, ab2r, vW1, vb1r, vW2p, vb2p)
    probs, v8 = _tc3(agg2_lo, agg2_hi, p2_lo, p2_hi, dinv, b2r, batch3, ws)
    return (probs, v8[:, :1])
